# baseline (speedup 1.0000x reference)
import functools

import jax
import jax.numpy as jnp
from jax import lax
from jax.experimental import pallas as pl
from jax.experimental.pallas import tpu as pltpu
from jax.experimental.pallas import tpu_sc as plsc

NUM_CORES = 2
NUM_SUBCORES = 16
NUM_WORKERS = NUM_CORES * NUM_SUBCORES
CHUNK = 128
LANES = 16


def _gather_lmajor(table, idxT, batch, hist):
    v, d = table.shape
    bw = batch // NUM_WORKERS
    assert bw % CHUNK == 0 and d % LANES == 0

    mesh = plsc.VectorSubcoreMesh(core_axis_name="c", subcore_axis_name="s")

    @functools.partial(
        pl.kernel,
        out_type=jax.ShapeDtypeStruct((hist, d, batch), jnp.float32),
        mesh=mesh,
        scratch_types=[
            pltpu.VMEM((hist * bw,), jnp.int32),
            pltpu.VMEM((2, bw, d), jnp.float32),
            pltpu.VMEM((2, d, bw), jnp.float32),
            pltpu.SemaphoreType.DMA,
            pltpu.SemaphoreType.DMA,
            pltpu.SemaphoreType.DMA,
            pltpu.SemaphoreType.DMA,
        ],
        compiler_params=pltpu.CompilerParams(
            use_tc_tiling_on_sc=False, needs_layout_passes=False),
    )
    def body(t_hbm, idx_hbm, out_hbm, idx_v, g_v, oblk_v,
             isem, gsem0, gsem1, osem):
        wid = lax.axis_index("s") * NUM_CORES + lax.axis_index("c")
        b0 = wid * bw

        for l in range(hist):
            pltpu.async_copy(
                idx_hbm.at[pl.ds(l * batch + b0, bw)],
                idx_v.at[pl.ds(l * bw, bw)],
                isem,
            )
        pltpu.make_async_copy(idx_hbm.at[pl.ds(0, hist * bw)], idx_v, isem
                              ).wait()

        def gather_descs(l, buf):
            sem = gsem0 if buf == 0 else gsem1
            return [
                pltpu.make_async_copy(
                    t_hbm.at[idx_v.at[pl.ds(l * bw + j * CHUNK, CHUNK)]],
                    g_v.at[buf, pl.ds(j * CHUNK, CHUNK)],
                    sem,
                )
                for j in range(bw // CHUNK)
            ]

        def out_desc(l, buf):
            return pltpu.make_async_copy(
                oblk_v.at[buf],
                out_hbm.at[l, :, pl.ds(b0, bw)],
                osem,
            )

        def transpose(buf):
            @plsc.parallel_loop(0, bw // LANES, 1, unroll=8)
            def _(c16):
                rowv = lax.iota(jnp.int32, LANES) + c16 * LANES
                for c in range(d):
                    colv = jnp.full((LANES,), c, jnp.int32)
                    vals = plsc.load_gather(g_v.at[buf], [rowv, colv])
                    oblk_v[buf, c, pl.ds(c16 * LANES, LANES)] = vals

        for dsc in gather_descs(0, 0):
            dsc.start()

        def step(l, buf):
            @pl.when(l + 1 < hist)
            def _():
                for dsc in gather_descs(l + 1, 1 - buf):
                    dsc.start()

            for dsc in gather_descs(0, buf):
                dsc.wait()

            @pl.when(l >= 2)
            def _():
                out_desc(0, buf).wait()

            transpose(buf)
            out_desc(l, buf).start()

        def pair_body(p, carry):
            step(2 * p, 0)
            step(2 * p + 1, 1)
            return carry

        lax.fori_loop(0, hist // 2, pair_body, 0)
        out_desc(0, 0).wait()
        out_desc(0, 1).wait()

    return body(table, idxT)


def kernel(go, emb_weights):
    b, h = go.shape
    idxT = jnp.maximum(go.T.reshape(-1).astype(jnp.int32), 0)
    out2 = _gather_lmajor(emb_weights, idxT, b, h)
    return jnp.transpose(out2, (2, 0, 1))


# speedup vs baseline: 1.5313x; 1.5313x over previous
import functools

import jax
import jax.numpy as jnp
from jax import lax
from jax.experimental import pallas as pl
from jax.experimental.pallas import tpu as pltpu
from jax.experimental.pallas import tpu_sc as plsc

NUM_CORES = 2
NUM_SUBCORES = 16
NUM_WORKERS = NUM_CORES * NUM_SUBCORES
CHUNK = 128
LANES = 16


def _gather_lmajor(table, idxT, batch, hist):
    v, d = table.shape
    bw = batch // NUM_WORKERS
    assert bw % CHUNK == 0 and d % LANES == 0

    mesh = plsc.VectorSubcoreMesh(core_axis_name="c", subcore_axis_name="s")

    @functools.partial(
        pl.kernel,
        out_type=jax.ShapeDtypeStruct((hist, d, batch), jnp.float32),
        mesh=mesh,
        scratch_types=[
            pltpu.VMEM((hist * bw,), jnp.int32),
            pltpu.VMEM((2, bw, d), jnp.float32),
            pltpu.VMEM((2, d, bw + 1), jnp.float32),
            pltpu.SemaphoreType.DMA,
            pltpu.SemaphoreType.DMA,
            pltpu.SemaphoreType.DMA,
            pltpu.SemaphoreType.DMA,
            pltpu.SemaphoreType.DMA,
        ],
        compiler_params=pltpu.CompilerParams(
            use_tc_tiling_on_sc=False, needs_layout_passes=False),
    )
    def body(t_hbm, idx_hbm, out_hbm, idx_v, g_v, oblk_v,
             isem, gsem0, gsem1, osem0, osem1):
        wid = lax.axis_index("s") * NUM_CORES + lax.axis_index("c")
        b0 = wid * bw

        for l in range(hist):
            pltpu.async_copy(
                idx_hbm.at[pl.ds(l * batch + b0, bw)],
                idx_v.at[pl.ds(l * bw, bw)],
                isem,
            )
        pltpu.make_async_copy(idx_hbm.at[pl.ds(0, hist * bw)], idx_v, isem
                              ).wait()

        def gather_descs(l, buf):
            sem = gsem0 if buf == 0 else gsem1
            return [
                pltpu.make_async_copy(
                    t_hbm.at[idx_v.at[pl.ds(l * bw + j * CHUNK, CHUNK)]],
                    g_v.at[buf, pl.ds(j * CHUNK, CHUNK)],
                    sem,
                )
                for j in range(bw // CHUNK)
            ]

        def out_descs(l, buf):
            sem = osem0 if buf == 0 else osem1
            return [
                pltpu.make_async_copy(
                    oblk_v.at[buf, c, pl.ds(0, bw)],
                    out_hbm.at[l, c, pl.ds(b0, bw)],
                    sem,
                )
                for c in range(d)
            ]

        def transpose(buf):
            c_lo = lax.iota(jnp.int32, LANES)
            c_hi = c_lo + LANES

            @plsc.parallel_loop(0, bw, 1, unroll=4)
            def _(bp):
                bsp = jnp.full((LANES,), bp, jnp.int32)
                v0 = g_v[buf, bp, pl.ds(0, LANES)]
                v1 = g_v[buf, bp, pl.ds(LANES, LANES)]
                plsc.store_scatter(oblk_v.at[buf], [c_lo, bsp], v0)
                plsc.store_scatter(oblk_v.at[buf], [c_hi, bsp], v1)

        for dsc in gather_descs(0, 0):
            dsc.start()

        def step(l, buf):
            @pl.when(l + 1 < hist)
            def _():
                for dsc in gather_descs(l + 1, 1 - buf):
                    dsc.start()

            for dsc in gather_descs(0, buf):
                dsc.wait()

            @pl.when(l >= 2)
            def _():
                for dsc in out_descs(0, buf):
                    dsc.wait()

            transpose(buf)
            for dsc in out_descs(l, buf):
                dsc.start()

        def pair_body(p, carry):
            step(2 * p, 0)
            step(2 * p + 1, 1)
            return carry

        lax.fori_loop(0, hist // 2, pair_body, 0)
        for dsc in out_descs(0, 0):
            dsc.wait()
        for dsc in out_descs(0, 1):
            dsc.wait()

    return body(table, idxT)


def kernel(go, emb_weights):
    b, h = go.shape
    idxT = jnp.maximum(go.T.reshape(-1).astype(jnp.int32), 0)
    out2 = _gather_lmajor(emb_weights, idxT, b, h)
    return jnp.transpose(out2, (2, 0, 1))
